# baseline (speedup 1.0000x reference)
import functools
import math

import jax
import jax.numpy as jnp
import numpy as np
from jax import lax
from jax.experimental import pallas as pl
from jax.experimental.pallas import tpu as pltpu
from jax.experimental.pallas import tpu_sc as plsc

_B, _T, _F, _E = 1024, 50, 26, 16
_D = _F * _E
_N = _B * _T * _F
_H = _N // 2
_SCALE = math.sqrt(float(_D))

_NW = 32
_PERIOD = _T * _F
_STEP = 1040
_SUB = 104
_NSUB = _STEP // _SUB
_NQ = 5
_BLK = _NQ * _STEP
_NBLK = _H // (_NW * _BLK)


def _pe_rows():
    pe = np.zeros((_T, _D), dtype=np.float32)
    pos = np.arange(_T, dtype=np.float32)[:, None]
    denom = np.exp(np.arange(0, _D, 2, dtype=np.float32) * (-np.log(10000.0) / _D))
    pe[:, 0::2] = np.sin(pos * denom)
    pe[:, 1::2] = np.cos(pos * denom)
    return jnp.asarray(pe.reshape(_PERIOD * _E))


def _round_pack(a, b):
    ua = lax.bitcast_convert_type(a, jnp.int32)
    ub = lax.bitcast_convert_type(b, jnp.int32)
    ua = ua + jnp.int32(0x7FFF) + (lax.shift_right_logical(ua, 16) & jnp.int32(1))
    ub = ub + jnp.int32(0x7FFF) + (lax.shift_right_logical(ub, 16) & jnp.int32(1))
    return lax.shift_right_logical(ua, 16) | (ub & jnp.int32(-65536))


def _sc_body(idx_s_hbm, idx_t_hbm, pe_hbm, table_hbm, raw_s_hbm, raw_t_hbm,
             pe_v, idx_a, idx_b, gbuf_a, gbuf_b, obuf0, obuf1,
             sem_g, sem_o0, sem_o1):
    wid = lax.axis_index("s") * 2 + lax.axis_index("c")
    pltpu.sync_copy(pe_hbm, pe_v)
    obufs = (obuf0, obuf1)
    sems_o = (sem_o0, sem_o1)

    for phase, (idx_hbm, raw_hbm) in enumerate(
            ((idx_s_hbm, raw_s_hbm), (idx_t_hbm, raw_t_hbm))):
        def block_body(b, carry, idx_hbm=idx_hbm, raw_hbm=raw_hbm,
                       phase=phase):
            blk = wid * _NBLK + b
            pltpu.sync_copy(idx_hbm.at[pl.ds(blk * _BLK, _BLK)], idx_a)
            pltpu.sync_copy(idx_hbm.at[pl.ds(_H + blk * _BLK, _BLK)], idx_b)

            for q in range(_NQ):
                s = phase * 20 + b * _NQ + q
                pp0 = (q * _STEP) % _PERIOD
                base = blk * _BLK + q * _STEP

                def fire(j, carry2, q=q):
                    pltpu.async_copy(
                        table_hbm.at[idx_a.at[pl.ds(q * _STEP + j * _SUB, _SUB)]],
                        gbuf_a.at[pl.ds(j * _SUB, _SUB)], sem_g)
                    pltpu.async_copy(
                        table_hbm.at[idx_b.at[pl.ds(q * _STEP + j * _SUB, _SUB)]],
                        gbuf_b.at[pl.ds(j * _SUB, _SUB)], sem_g)
                    return carry2
                lax.fori_loop(0, _NSUB, fire, 0)
                pltpu.make_async_copy(table_hbm.at[pl.ds(0, _STEP)], gbuf_a,
                                      sem_g).wait()
                pltpu.make_async_copy(table_hbm.at[pl.ds(0, _STEP)], gbuf_b,
                                      sem_g).wait()

                obuf = obufs[q % 2]
                sem_o = sems_o[q % 2]
                @pl.when(s >= 2)
                def _():
                    pltpu.make_async_copy(
                        obuf, raw_hbm.at[pl.ds(0, _STEP * _E)], sem_o).wait()

                def comp(i, carry2, obuf=obuf, pp0=pp0):
                    t = pp0 + i
                    t = jnp.where(t >= _PERIOD, t - _PERIOD, t)
                    p = pe_v[pl.ds(t * _E, _E)]
                    a = gbuf_a[i] * _SCALE + p
                    bb = gbuf_b[i] * _SCALE + p
                    obuf[pl.ds(i * _E, _E)] = _round_pack(a, bb)
                    return carry2
                lax.fori_loop(0, _STEP, comp, 0)

                pltpu.async_copy(obuf, raw_hbm.at[pl.ds(base * _E, _STEP * _E)],
                                 sem_o)
            return carry

        lax.fori_loop(0, _NBLK, block_body, 0)

    pltpu.make_async_copy(obuf0, raw_t_hbm.at[pl.ds(0, _STEP * _E)],
                          sem_o0).wait()
    pltpu.make_async_copy(obuf1, raw_t_hbm.at[pl.ds(0, _STEP * _E)],
                          sem_o1).wait()


def _make_call():
    mesh = plsc.VectorSubcoreMesh(core_axis_name="c", subcore_axis_name="s")
    return functools.partial(
        pl.kernel,
        mesh=mesh,
        compiler_params=pltpu.CompilerParams(use_tc_tiling_on_sc=False),
        out_type=[jax.ShapeDtypeStruct((_H * _E,), jnp.int32),
                  jax.ShapeDtypeStruct((_H * _E,), jnp.int32)],
        scratch_types=[
            pltpu.VMEM((_PERIOD * _E,), jnp.float32),
            pltpu.VMEM((_BLK,), jnp.int32),
            pltpu.VMEM((_BLK,), jnp.int32),
            pltpu.VMEM((_STEP, _E), jnp.float32),
            pltpu.VMEM((_STEP, _E), jnp.float32),
            pltpu.VMEM((_STEP * _E,), jnp.int32),
            pltpu.VMEM((_STEP * _E,), jnp.int32),
            pltpu.SemaphoreType.DMA,
            pltpu.SemaphoreType.DMA,
            pltpu.SemaphoreType.DMA,
        ],
    )(_sc_body)


def _expand(raw):
    pairs = lax.bitcast_convert_type(raw, jnp.bfloat16)
    lo = pairs[:, 0].astype(jnp.float32)
    hi = pairs[:, 1].astype(jnp.float32)
    return jnp.concatenate([lo, hi], axis=0)


def kernel(src, tgt, table):
    pe = _pe_rows()
    f = _make_call()
    raw_s, raw_t = f(src.reshape(_N), tgt.reshape(_N), pe, table)
    out_s = _expand(raw_s).reshape(_B, _T, _D)
    out_t = _expand(raw_t).reshape(_B, _T, _D)
    return out_s, out_t


# speedup vs baseline: 1.8795x; 1.8795x over previous
import functools
import math

import jax
import jax.numpy as jnp
import numpy as np
from jax import lax
from jax.experimental import pallas as pl
from jax.experimental.pallas import tpu as pltpu
from jax.experimental.pallas import tpu_sc as plsc

_B, _T, _F, _E = 1024, 50, 26, 16
_D = _F * _E
_N = _B * _T * _F
_SCALE = math.sqrt(float(_D))

_NW = 32
_PERIOD = _T * _F
_STEP = 1040
_SUB = 104
_NSUB = _STEP // _SUB
_NQ = 5
_BLK = _NQ * _STEP
_NBLK = _N // (_NW * _BLK)
_PAIRS = _STEP // 2


def _pe_rows():
    pe = np.zeros((_T, _D), dtype=np.float32)
    pos = np.arange(_T, dtype=np.float32)[:, None]
    denom = np.exp(np.arange(0, _D, 2, dtype=np.float32) * (-np.log(10000.0) / _D))
    pe[:, 0::2] = np.sin(pos * denom)
    pe[:, 1::2] = np.cos(pos * denom)
    flat = pe.reshape(_PERIOD * _E)
    return jnp.asarray(np.concatenate([flat, flat]))


def _round_lo(x):
    u = lax.bitcast_convert_type(x, jnp.int32)
    u = u + jnp.int32(0x7FFF) + (lax.shift_right_logical(u, 16) & jnp.int32(1))
    return lax.shift_right_logical(u, 16)


def _sc_body(idx_s_hbm, idx_t_hbm, pe_hbm, table_hbm, raw_s_hbm, raw_t_hbm,
             pe_v, idx_v, gbuf, obuf0, obuf1, sem_g, sem_o0, sem_o1):
    wid = lax.axis_index("s") * 2 + lax.axis_index("c")
    pltpu.sync_copy(pe_hbm, pe_v)
    obufs = (obuf0, obuf1)
    sems_o = (sem_o0, sem_o1)

    lane = lax.iota(jnp.int32, 16)
    evens = (lane * 2) & jnp.int32(15)
    odds = evens + 1
    first_half = lane < 8

    def pack_pair(a, b):
        a_e = a.at[evens].get(mode="promise_in_bounds")
        b_e = b.at[evens].get(mode="promise_in_bounds")
        a_o = a.at[odds].get(mode="promise_in_bounds")
        b_o = b.at[odds].get(mode="promise_in_bounds")
        even = jnp.where(first_half, a_e, b_e)
        odd = jnp.where(first_half, a_o, b_o)
        return even | lax.shift_left(odd, 16)

    for phase, (idx_hbm, raw_hbm) in enumerate(
            ((idx_s_hbm, raw_s_hbm), (idx_t_hbm, raw_t_hbm))):
        def block_body(b, carry, idx_hbm=idx_hbm, raw_hbm=raw_hbm,
                       phase=phase):
            blk = wid * _NBLK + b
            pltpu.sync_copy(idx_hbm.at[pl.ds(blk * _BLK, _BLK)], idx_v)

            for q in range(_NQ):
                s = phase * (_NBLK * _NQ) + b * _NQ + q
                pp0 = (q * _STEP) % _PERIOD
                base = blk * _BLK + q * _STEP

                def fire(j, carry2, q=q):
                    pltpu.async_copy(
                        table_hbm.at[idx_v.at[pl.ds(q * _STEP + j * _SUB, _SUB)]],
                        gbuf.at[pl.ds(j * _SUB, _SUB)], sem_g)
                    return carry2
                lax.fori_loop(0, _NSUB, fire, 0)
                pltpu.make_async_copy(table_hbm.at[pl.ds(0, _STEP)], gbuf,
                                      sem_g).wait()

                obuf = obufs[q % 2]
                sem_o = sems_o[q % 2]
                @pl.when(s >= 2)
                def _():
                    pltpu.make_async_copy(
                        obuf, raw_hbm.at[pl.ds(0, _PAIRS * _E)], sem_o).wait()

                def comp(k, carry2, obuf=obuf, pp0=pp0):
                    p0 = pe_v[pl.ds((pp0 + 2 * k) * _E, _E)]
                    p1 = pe_v[pl.ds((pp0 + 2 * k) * _E + _E, _E)]
                    a = _round_lo(gbuf[2 * k] * _SCALE + p0)
                    bb = _round_lo(gbuf[2 * k + 1] * _SCALE + p1)
                    obuf[pl.ds(k * _E, _E)] = pack_pair(a, bb)
                    return carry2
                lax.fori_loop(0, _PAIRS, comp, 0)

                pltpu.async_copy(
                    obuf, raw_hbm.at[pl.ds(base // 2 * _E, _PAIRS * _E)], sem_o)
            return carry

        lax.fori_loop(0, _NBLK, block_body, 0)

    pltpu.make_async_copy(obuf0, raw_t_hbm.at[pl.ds(0, _PAIRS * _E)],
                          sem_o0).wait()
    pltpu.make_async_copy(obuf1, raw_t_hbm.at[pl.ds(0, _PAIRS * _E)],
                          sem_o1).wait()


def _make_call():
    mesh = plsc.VectorSubcoreMesh(core_axis_name="c", subcore_axis_name="s")
    return functools.partial(
        pl.kernel,
        mesh=mesh,
        compiler_params=pltpu.CompilerParams(use_tc_tiling_on_sc=False),
        out_type=[jax.ShapeDtypeStruct((_N // 2 * _E,), jnp.int32),
                  jax.ShapeDtypeStruct((_N // 2 * _E,), jnp.int32)],
        scratch_types=[
            pltpu.VMEM((2 * _PERIOD * _E,), jnp.float32),
            pltpu.VMEM((_BLK,), jnp.int32),
            pltpu.VMEM((_STEP, _E), jnp.float32),
            pltpu.VMEM((_PAIRS * _E,), jnp.int32),
            pltpu.VMEM((_PAIRS * _E,), jnp.int32),
            pltpu.SemaphoreType.DMA,
            pltpu.SemaphoreType.DMA,
            pltpu.SemaphoreType.DMA,
        ],
    )(_sc_body)


def _expand(raw):
    bf = lax.bitcast_convert_type(raw, jnp.bfloat16)
    return bf.astype(jnp.float32).reshape(_B, _T, _D)


def kernel(src, tgt, table):
    pe = _pe_rows()
    f = _make_call()
    raw_s, raw_t = f(src.reshape(_N), tgt.reshape(_N), pe, table)
    return _expand(raw_s), _expand(raw_t)


# speedup vs baseline: 2.5542x; 1.3590x over previous
import functools
import math

import jax
import jax.numpy as jnp
import numpy as np
from jax import lax
from jax.experimental import pallas as pl
from jax.experimental.pallas import tpu as pltpu
from jax.experimental.pallas import tpu_sc as plsc

_B, _T, _F, _E = 1024, 50, 26, 16
_D = _F * _E
_N = _B * _T * _F
_V = 1000000
_SCALE = math.sqrt(float(_D))

_NW = 32
_PERIOD = _T * _F
_SUB = 128
_RPS = 13
_STEP = _RPS * _SUB
_NSTEP = _N // (_NW * _STEP)


def _pe_rows():
    pe = np.zeros((_T, _D), dtype=np.float32)
    pos = np.arange(_T, dtype=np.float32)[:, None]
    denom = np.exp(np.arange(0, _D, 2, dtype=np.float32) * (-np.log(10000.0) / _D))
    pe[:, 0::2] = np.sin(pos * denom)
    pe[:, 1::2] = np.cos(pos * denom)
    flat = pe.reshape(_PERIOD * _E)
    return jnp.asarray(np.concatenate([flat, flat, flat]))


def _sc_body(idx_s_hbm, idx_t_hbm, pe_hbm, table_hbm, out_s_hbm, out_t_hbm,
             pe_v, idx_v, gbuf, sem_g):
    wid = lax.axis_index("s") * 2 + lax.axis_index("c")
    pltpu.sync_copy(pe_hbm, pe_v)
    table_r = table_hbm

    for idx_hbm, out_hbm in ((idx_s_hbm, out_s_hbm), (idx_t_hbm, out_t_hbm)):
        def step_body(s, carry, idx_hbm=idx_hbm, out_hbm=out_hbm):
            r0 = wid * (_NSTEP * _RPS) + s * _RPS
            base = r0 * _SUB
            pltpu.sync_copy(idx_hbm.at[pl.ds(r0, _RPS)], idx_v)

            def fire(j, carry2):
                pltpu.async_copy(table_r.at[idx_v.at[j]],
                                 gbuf.at[pl.ds(j * _SUB, _SUB)], sem_g)
                return carry2
            lax.fori_loop(0, _RPS, fire, 0)
            pltpu.make_async_copy(table_r.at[pl.ds(0, _STEP)], gbuf,
                                  sem_g).wait()

            pp0 = lax.rem(base, _PERIOD)
            def comp(i, carry2):
                t = pp0 + 2 * i
                a = gbuf[2 * i] * _SCALE + pe_v[pl.ds(t * _E, _E)]
                b = gbuf[2 * i + 1] * _SCALE + pe_v[pl.ds(t * _E + _E, _E)]
                gbuf[2 * i] = a
                gbuf[2 * i + 1] = b
                return carry2
            lax.fori_loop(0, _STEP // 2, comp, 0)

            pltpu.sync_copy(gbuf, out_hbm.at[pl.ds(base, _STEP)])
            return carry
        lax.fori_loop(0, _NSTEP, step_body, 0)


def kernel(src, tgt, table):
    pe = _pe_rows()
    mesh = plsc.VectorSubcoreMesh(core_axis_name="c", subcore_axis_name="s")
    f = functools.partial(
        pl.kernel,
        mesh=mesh,
        compiler_params=pltpu.CompilerParams(use_tc_tiling_on_sc=False),
        out_type=[jax.ShapeDtypeStruct((_N, _E), jnp.float32),
                  jax.ShapeDtypeStruct((_N, _E), jnp.float32)],
        scratch_types=[
            pltpu.VMEM((3 * _PERIOD * _E,), jnp.float32),
            pltpu.VMEM((_RPS, _SUB), jnp.int32),
            pltpu.VMEM((_STEP, _E), jnp.float32),
            pltpu.SemaphoreType.DMA,
        ],
    )(_sc_body)
    out_s, out_t = f(src.reshape(_N // _SUB, _SUB), tgt.reshape(_N // _SUB, _SUB),
                     pe, table)
    return out_s.reshape(_B, _T, _D), out_t.reshape(_B, _T, _D)


# speedup vs baseline: 3.0498x; 1.1940x over previous
import functools
import math

import jax
import jax.numpy as jnp
import numpy as np
from jax import lax
from jax.experimental import pallas as pl
from jax.experimental.pallas import tpu as pltpu
from jax.experimental.pallas import tpu_sc as plsc

_B, _T, _F, _E = 1024, 50, 26, 16
_D = _F * _E
_N = _B * _T * _F
_SCALE = math.sqrt(float(_D))

_NW = 32
_PERIOD = _T * _F
_CHUNK_B = 4
_CHUNK = _CHUNK_B * _PERIOD
_SUB = 100
_NSUB = _CHUNK // _SUB
_NCHUNK = _N // (_NW * _CHUNK)


def _pe_rows():
    pe = np.zeros((_T, _D), dtype=np.float32)
    pos = np.arange(_T, dtype=np.float32)[:, None]
    denom = np.exp(np.arange(0, _D, 2, dtype=np.float32) * (-np.log(10000.0) / _D))
    pe[:, 0::2] = np.sin(pos * denom)
    pe[:, 1::2] = np.cos(pos * denom)
    return jnp.asarray(pe.reshape(_PERIOD, _E))


def _sc_body(src_hbm, tgt_hbm, pe_hbm, table_hbm, out_src, out_tgt,
             pe_v, idx_v, rows_v, sem):
    wid = lax.axis_index("s") * 2 + lax.axis_index("c")
    pltpu.sync_copy(pe_hbm, pe_v)

    for idx_hbm, out_hbm in ((src_hbm, out_src), (tgt_hbm, out_tgt)):
        def chunk_body(c, carry, idx_hbm=idx_hbm, out_hbm=out_hbm):
            cid = wid * _NCHUNK + c
            n0 = cid * _CHUNK
            pltpu.sync_copy(idx_hbm.at[cid], idx_v)

            def fire(j, carry2):
                pltpu.async_copy(table_hbm.at[idx_v.at[j]],
                                 rows_v.at[pl.ds(j * _SUB, _SUB)], sem)
                return carry2
            lax.fori_loop(0, _NSUB, fire, 0)
            pltpu.make_async_copy(out_hbm.at[pl.ds(0, _CHUNK)], rows_v,
                                  sem).wait()

            def comp(i, carry2):
                p = pe_v[i]
                for h in range(_CHUNK_B):
                    r = rows_v[h * _PERIOD + i]
                    rows_v[h * _PERIOD + i] = r * _SCALE + p
                return carry2
            lax.fori_loop(0, _PERIOD, comp, 0)

            pltpu.sync_copy(rows_v, out_hbm.at[pl.ds(n0, _CHUNK)])
            return carry
        lax.fori_loop(0, _NCHUNK, chunk_body, 0)


def kernel(src, tgt, table):
    src_i = src.reshape(_N // _CHUNK, _NSUB, _SUB)
    tgt_i = tgt.reshape(_N // _CHUNK, _NSUB, _SUB)
    pe = _pe_rows()

    mesh = plsc.VectorSubcoreMesh(core_axis_name="c", subcore_axis_name="s")
    f = functools.partial(
        pl.kernel,
        mesh=mesh,
        compiler_params=pltpu.CompilerParams(use_tc_tiling_on_sc=False),
        out_type=[jax.ShapeDtypeStruct((_N, _E), jnp.float32),
                  jax.ShapeDtypeStruct((_N, _E), jnp.float32)],
        scratch_types=[
            pltpu.VMEM((_PERIOD, _E), jnp.float32),
            pltpu.VMEM((_NSUB, _SUB), jnp.int32),
            pltpu.VMEM((_CHUNK, _E), jnp.float32),
            pltpu.SemaphoreType.DMA,
        ],
    )(_sc_body)
    out_s, out_t = f(src_i, tgt_i, pe, table)
    return out_s.reshape(_B, _T, _D), out_t.reshape(_B, _T, _D)
